# baseline (speedup 1.0000x reference)
import functools

import jax
import jax.numpy as jnp
from jax import lax
from jax.experimental import pallas as pl
from jax.experimental.pallas import tpu as pltpu
from jax.experimental.pallas import tpu_sc as plsc

N = 10000
D = 128
E = 320000
NC = 2
NS = 16
NW = NC * NS
CHUNK = 128
RING = 2
IB = 8
NCH0 = 80
NCH1 = 80
NBLK0 = NCH0 // IB
NBLK1 = NCH1 // IB
EPW0 = NCH0 * CHUNK
EPW1 = NCH1 * CHUNK
EP = NS * (EPW0 + EPW1)
NPAD = 10112
RT = NPAD // NS
DEGW = 16


def _zero_vmem_rows(ref, rows, width):
    z = jnp.zeros((16,), jnp.float32)

    def body(i, _):
        for j in range(width // 16):
            ref[i, pl.ds(j * 16, 16)] = z
        return 0

    lax.fori_loop(0, rows, body, 0, unroll=False)


def _make_sc_segsum(with_deg):
    out_type = [jax.ShapeDtypeStruct((NC, NPAD, D), jnp.float32)]
    scratch = (
        [pltpu.VMEM((IB, CHUNK), jnp.int32),
         pltpu.VMEM((IB, CHUNK), jnp.int32)]
        + [pltpu.VMEM((CHUNK, D), jnp.float32)] * RING
        + [pltpu.VMEM_SHARED((NPAD, D), jnp.float32)]
        + [pltpu.SemaphoreType.DMA] * RING
    )
    if with_deg:
        out_type.append(jax.ShapeDtypeStruct((NC, NPAD, DEGW), jnp.float32))
        scratch += [
            pltpu.VMEM((CHUNK, DEGW), jnp.float32),
            pltpu.VMEM((CHUNK, DEGW), jnp.float32),
            pltpu.VMEM_SHARED((NPAD, DEGW), jnp.float32),
        ]
    mesh = plsc.VectorSubcoreMesh(core_axis_name="c", subcore_axis_name="s")

    def body(y, srcs, dsts, *refs):
        if with_deg:
            (acc_out, deg_out, idx_s, idx_d, *rest) = refs
            bufs, rest = rest[:RING], rest[RING:]
            (acc_sh, *sems, ones_v, zdeg, deg_sh) = rest
        else:
            (acc_out, idx_s, idx_d, *rest) = refs
            bufs, rest = rest[:RING], rest[RING:]
            (acc_sh, *sems) = rest
        buf = bufs[0]
        cid = lax.axis_index("c")
        sid = lax.axis_index("s")
        wid = cid * NS + sid
        nblk = jnp.where(cid == 0, NBLK0, NBLK1)

        _zero_vmem_rows(buf, CHUNK, D)
        r0 = sid * RT
        done = 0
        for t in range((RT + CHUNK - 1) // CHUNK):
            rows = min(CHUNK, RT - done)
            pltpu.sync_copy(buf.at[pl.ds(0, rows)],
                            acc_sh.at[pl.ds(r0 + done, rows)])
            done += rows
        if with_deg:
            one = jnp.ones((16,), jnp.float32)

            def fill_ones(i, _):
                ones_v[i] = one
                return 0

            lax.fori_loop(0, CHUNK, fill_ones, 0, unroll=False)
            _zero_vmem_rows(zdeg, CHUNK, DEGW)
            done = 0
            for t in range((RT + CHUNK - 1) // CHUNK):
                rows = min(CHUNK, RT - done)
                pltpu.sync_copy(zdeg.at[pl.ds(0, rows)],
                                deg_sh.at[pl.ds(r0 + done, rows)])
                done += rows
        plsc.subcore_barrier()

        def fire(l, j):
            pltpu.async_copy(y.at[idx_s.at[j]], bufs[l], sems[l])

        def wait(l, j):
            pltpu.make_async_copy(y.at[idx_s.at[j]], bufs[l],
                                  sems[l]).wait()

        def scat(l, j):
            pltpu.sync_copy(bufs[l], acc_sh.at[idx_d.at[j]], add=True)
            if with_deg:
                pltpu.sync_copy(ones_v, deg_sh.at[idx_d.at[j]], add=True)

        def run_block():
            for l in range(RING - 1):
                fire(l, l)

            def group(g, _):
                for l in range(RING):
                    j = RING * g + l
                    wait(l, j)
                    fire((l + RING - 1) % RING, j + RING - 1)
                    scat(l, j)
                return 0

            lax.fori_loop(0, IB // RING - 1, group, 0, unroll=False)
            for l in range(RING):
                j = IB - RING + l
                wait(l, j)
                if j + RING - 1 < IB:
                    fire((l + RING - 1) % RING, j + RING - 1)
                scat(l, j)

        def block(blk, _):
            pltpu.sync_copy(srcs.at[wid, pl.ds(blk * IB, IB)],
                            idx_s)
            pltpu.sync_copy(dsts.at[wid, pl.ds(blk * IB, IB)],
                            idx_d)
            run_block()
            return 0

        lax.fori_loop(0, nblk, block, 0, unroll=False)
        plsc.subcore_barrier()

        pltpu.sync_copy(acc_sh.at[pl.ds(r0, RT)],
                        acc_out.at[cid, pl.ds(r0, RT)])
        if with_deg:
            pltpu.sync_copy(deg_sh.at[pl.ds(r0, RT)],
                            deg_out.at[cid, pl.ds(r0, RT)])

    return pl.kernel(body, out_type=out_type, mesh=mesh,
                     scratch_types=scratch,
                     compiler_params=pltpu.CompilerParams(
                         use_tc_tiling_on_sc=False))


_sc_segsum_deg = _make_sc_segsum(with_deg=True)
_sc_segsum = _make_sc_segsum(with_deg=False)

BM = 1000
_DN = (((1,), (1,)), ((), ()))


def _tc_pre_body(x_ref, wl_ref, wr_ref, b_ref, y_ref, z_ref):
    xb = x_ref[...]
    y_ref[...] = lax.dot_general(xb, wl_ref[...], _DN,
                                 preferred_element_type=jnp.float32)
    z_ref[...] = lax.dot_general(xb, wr_ref[...], _DN,
                                 preferred_element_type=jnp.float32) + b_ref[...]


_tc_pre = pl.pallas_call(
    _tc_pre_body,
    grid=(N // BM,),
    in_specs=[
        pl.BlockSpec((BM, D), lambda m: (m, 0)),
        pl.BlockSpec((D, D), lambda m: (0, 0)),
        pl.BlockSpec((D, D), lambda m: (0, 0)),
        pl.BlockSpec((1, D), lambda m: (0, 0)),
    ],
    out_specs=[pl.BlockSpec((BM, D), lambda m: (m, 0)),
               pl.BlockSpec((BM, D), lambda m: (m, 0))],
    out_shape=[jax.ShapeDtypeStruct((N, D), jnp.float32)] * 2,
)


def _tc_mid_body(acc_ref, deg_ref, z1_ref, wl_ref, wr_ref, b_ref,
                 y_ref, z_ref):
    agg = acc_ref[0] + acc_ref[1]
    d = deg_ref[0, :, 0:1] + deg_ref[1, :, 0:1]
    inv = 1.0 / jnp.maximum(d, 1.0)
    h = jnp.maximum(agg * inv + z1_ref[...], 0.0)
    y_ref[...] = lax.dot_general(h, wl_ref[...], _DN,
                                 preferred_element_type=jnp.float32)
    z_ref[...] = lax.dot_general(h, wr_ref[...], _DN,
                                 preferred_element_type=jnp.float32) + b_ref[...]


_tc_mid = pl.pallas_call(
    _tc_mid_body,
    grid=(N // BM,),
    in_specs=[
        pl.BlockSpec((NC, BM, D), lambda m: (0, m, 0)),
        pl.BlockSpec((NC, BM, DEGW), lambda m: (0, m, 0)),
        pl.BlockSpec((BM, D), lambda m: (m, 0)),
        pl.BlockSpec((D, D), lambda m: (0, 0)),
        pl.BlockSpec((D, D), lambda m: (0, 0)),
        pl.BlockSpec((1, D), lambda m: (0, 0)),
    ],
    out_specs=[pl.BlockSpec((BM, D), lambda m: (m, 0)),
               pl.BlockSpec((BM, D), lambda m: (m, 0))],
    out_shape=[jax.ShapeDtypeStruct((N, D), jnp.float32)] * 2,
)


def _tc_fin_body(acc_ref, deg_ref, z2_ref, out_ref):
    agg = acc_ref[0] + acc_ref[1]
    d = deg_ref[0, :, 0:1] + deg_ref[1, :, 0:1]
    inv = 1.0 / jnp.maximum(d, 1.0)
    out_ref[...] = agg * inv + z2_ref[...]


_tc_fin = pl.pallas_call(
    _tc_fin_body,
    grid=(N // BM,),
    in_specs=[
        pl.BlockSpec((NC, BM, D), lambda m: (0, m, 0)),
        pl.BlockSpec((NC, BM, DEGW), lambda m: (0, m, 0)),
        pl.BlockSpec((BM, D), lambda m: (m, 0)),
    ],
    out_specs=pl.BlockSpec((BM, D), lambda m: (m, 0)),
    out_shape=jax.ShapeDtypeStruct((N, D), jnp.float32),
)


def kernel(x, edge_index, edge_weight, W1l, b1, W1r, W2l, b2, W2r):
    del edge_weight
    src = edge_index[0].astype(jnp.int32)
    dst = edge_index[1].astype(jnp.int32)
    pad = EP - E
    src = jnp.concatenate([src, jnp.zeros((pad,), jnp.int32)])
    dst = jnp.concatenate([dst, jnp.full((pad,), N, jnp.int32)])

    nmax = max(NCH0, NCH1)

    def split(a):
        a0 = a[:NS * EPW0].reshape(NS, NCH0, CHUNK)
        a1 = a[NS * EPW0:].reshape(NS, NCH1, CHUNK)
        a0 = jnp.pad(a0, ((0, 0), (0, nmax - NCH0), (0, 0)))
        a1 = jnp.pad(a1, ((0, 0), (0, nmax - NCH1), (0, 0)))
        return jnp.concatenate([a0, a1], axis=0)

    src = split(src)
    dst = split(dst)
    b1r = b1.reshape(1, D)
    b2r = b2.reshape(1, D)

    y1, z1 = _tc_pre(x, W1l, W1r, b1r)
    acc1, deg = _sc_segsum_deg(y1, src, dst)
    y2, z2 = _tc_mid(acc1, deg, z1, W2l, W2r, b2r)
    acc2 = _sc_segsum(y2, src, dst)
    if isinstance(acc2, (list, tuple)):
        acc2 = acc2[0]
    return _tc_fin(acc2, deg, z2)


# speedup vs baseline: 1.1129x; 1.1129x over previous
import functools

import jax
import jax.numpy as jnp
from jax import lax
from jax.experimental import pallas as pl
from jax.experimental.pallas import tpu as pltpu
from jax.experimental.pallas import tpu_sc as plsc

N = 10000
D = 128
E = 320000
NC = 2
NS = 16
NW = NC * NS
CHUNK = 128
RING = 2
IB = 8
NCH0 = 104
NCH1 = 56
NBLK0 = NCH0 // IB
NBLK1 = NCH1 // IB
EPW0 = NCH0 * CHUNK
EPW1 = NCH1 * CHUNK
EP = NS * (EPW0 + EPW1)
NPAD = 10112
RT = NPAD // NS
DEGW = 16


def _zero_vmem_rows(ref, rows, width):
    z = jnp.zeros((16,), jnp.float32)

    def body(i, _):
        for j in range(width // 16):
            ref[i, pl.ds(j * 16, 16)] = z
        return 0

    lax.fori_loop(0, rows, body, 0, unroll=False)


def _make_sc_segsum(with_deg):
    out_type = [jax.ShapeDtypeStruct((NC, NPAD, D), jnp.float32)]
    scratch = (
        [pltpu.VMEM((IB, CHUNK), jnp.int32),
         pltpu.VMEM((IB, CHUNK), jnp.int32)]
        + [pltpu.VMEM((CHUNK, D), jnp.float32)] * RING
        + [pltpu.VMEM_SHARED((NPAD, D), jnp.float32)]
        + [pltpu.SemaphoreType.DMA] * RING
    )
    if with_deg:
        out_type.append(jax.ShapeDtypeStruct((NC, NPAD, DEGW), jnp.float32))
        scratch += [
            pltpu.VMEM((CHUNK, DEGW), jnp.float32),
            pltpu.VMEM((CHUNK, DEGW), jnp.float32),
            pltpu.VMEM_SHARED((NPAD, DEGW), jnp.float32),
        ]
    mesh = plsc.VectorSubcoreMesh(core_axis_name="c", subcore_axis_name="s")

    def body(y, srcs, dsts, *refs):
        if with_deg:
            (acc_out, deg_out, idx_s, idx_d, *rest) = refs
            bufs, rest = rest[:RING], rest[RING:]
            (acc_sh, *sems, ones_v, zdeg, deg_sh) = rest
        else:
            (acc_out, idx_s, idx_d, *rest) = refs
            bufs, rest = rest[:RING], rest[RING:]
            (acc_sh, *sems) = rest
        buf = bufs[0]
        cid = lax.axis_index("c")
        sid = lax.axis_index("s")
        wid = cid * NS + sid

        _zero_vmem_rows(buf, CHUNK, D)
        r0 = sid * RT
        done = 0
        for t in range((RT + CHUNK - 1) // CHUNK):
            rows = min(CHUNK, RT - done)
            pltpu.sync_copy(buf.at[pl.ds(0, rows)],
                            acc_sh.at[pl.ds(r0 + done, rows)])
            done += rows
        if with_deg:
            one = jnp.ones((16,), jnp.float32)

            def fill_ones(i, _):
                ones_v[i] = one
                return 0

            lax.fori_loop(0, CHUNK, fill_ones, 0, unroll=False)
            _zero_vmem_rows(zdeg, CHUNK, DEGW)
            done = 0
            for t in range((RT + CHUNK - 1) // CHUNK):
                rows = min(CHUNK, RT - done)
                pltpu.sync_copy(zdeg.at[pl.ds(0, rows)],
                                deg_sh.at[pl.ds(r0 + done, rows)])
                done += rows
        plsc.subcore_barrier()

        def fire(l, j):
            pltpu.async_copy(y.at[idx_s.at[j]], bufs[l], sems[l])

        def wait(l, j):
            pltpu.make_async_copy(y.at[idx_s.at[j]], bufs[l],
                                  sems[l]).wait()

        def scat(l, j):
            pltpu.sync_copy(bufs[l], acc_sh.at[idx_d.at[j]], add=True)
            if with_deg:
                pltpu.sync_copy(ones_v, deg_sh.at[idx_d.at[j]], add=True)

        def run_block():
            for l in range(RING - 1):
                fire(l, l)

            def group(g, _):
                for l in range(RING):
                    j = RING * g + l
                    wait(l, j)
                    fire((l + RING - 1) % RING, j + RING - 1)
                    scat(l, j)
                return 0

            lax.fori_loop(0, IB // RING - 1, group, 0, unroll=False)
            for l in range(RING):
                j = IB - RING + l
                wait(l, j)
                if j + RING - 1 < IB:
                    fire((l + RING - 1) % RING, j + RING - 1)
                scat(l, j)

        def block(blk):
            pltpu.sync_copy(srcs.at[wid, pl.ds(blk * IB, IB)],
                            idx_s)
            pltpu.sync_copy(dsts.at[wid, pl.ds(blk * IB, IB)],
                            idx_d)
            run_block()

        @pl.when(cid == 0)
        def _():
            for blk in range(NBLK0):
                block(blk)

        @pl.when(cid == 1)
        def _():
            for blk in range(NBLK1):
                block(blk)

        plsc.subcore_barrier()

        pltpu.sync_copy(acc_sh.at[pl.ds(r0, RT)],
                        acc_out.at[cid, pl.ds(r0, RT)])
        if with_deg:
            pltpu.sync_copy(deg_sh.at[pl.ds(r0, RT)],
                            deg_out.at[cid, pl.ds(r0, RT)])

    return pl.kernel(body, out_type=out_type, mesh=mesh,
                     scratch_types=scratch,
                     compiler_params=pltpu.CompilerParams(
                         use_tc_tiling_on_sc=False))


_sc_segsum_deg = _make_sc_segsum(with_deg=True)
_sc_segsum = _make_sc_segsum(with_deg=False)

BM = 1000
_DN = (((1,), (1,)), ((), ()))


def _tc_pre_body(x_ref, wl_ref, wr_ref, b_ref, y_ref, z_ref):
    xb = x_ref[...]
    y_ref[...] = lax.dot_general(xb, wl_ref[...], _DN,
                                 preferred_element_type=jnp.float32)
    z_ref[...] = lax.dot_general(xb, wr_ref[...], _DN,
                                 preferred_element_type=jnp.float32) + b_ref[...]


_tc_pre = pl.pallas_call(
    _tc_pre_body,
    grid=(N // BM,),
    in_specs=[
        pl.BlockSpec((BM, D), lambda m: (m, 0)),
        pl.BlockSpec((D, D), lambda m: (0, 0)),
        pl.BlockSpec((D, D), lambda m: (0, 0)),
        pl.BlockSpec((1, D), lambda m: (0, 0)),
    ],
    out_specs=[pl.BlockSpec((BM, D), lambda m: (m, 0)),
               pl.BlockSpec((BM, D), lambda m: (m, 0))],
    out_shape=[jax.ShapeDtypeStruct((N, D), jnp.float32)] * 2,
)


def _tc_mid_body(acc_ref, deg_ref, z1_ref, wl_ref, wr_ref, b_ref,
                 y_ref, z_ref):
    agg = acc_ref[0] + acc_ref[1]
    d = deg_ref[0, :, 0:1] + deg_ref[1, :, 0:1]
    inv = 1.0 / jnp.maximum(d, 1.0)
    h = jnp.maximum(agg * inv + z1_ref[...], 0.0)
    y_ref[...] = lax.dot_general(h, wl_ref[...], _DN,
                                 preferred_element_type=jnp.float32)
    z_ref[...] = lax.dot_general(h, wr_ref[...], _DN,
                                 preferred_element_type=jnp.float32) + b_ref[...]


_tc_mid = pl.pallas_call(
    _tc_mid_body,
    grid=(N // BM,),
    in_specs=[
        pl.BlockSpec((NC, BM, D), lambda m: (0, m, 0)),
        pl.BlockSpec((NC, BM, DEGW), lambda m: (0, m, 0)),
        pl.BlockSpec((BM, D), lambda m: (m, 0)),
        pl.BlockSpec((D, D), lambda m: (0, 0)),
        pl.BlockSpec((D, D), lambda m: (0, 0)),
        pl.BlockSpec((1, D), lambda m: (0, 0)),
    ],
    out_specs=[pl.BlockSpec((BM, D), lambda m: (m, 0)),
               pl.BlockSpec((BM, D), lambda m: (m, 0))],
    out_shape=[jax.ShapeDtypeStruct((N, D), jnp.float32)] * 2,
)


def _tc_fin_body(acc_ref, deg_ref, z2_ref, out_ref):
    agg = acc_ref[0] + acc_ref[1]
    d = deg_ref[0, :, 0:1] + deg_ref[1, :, 0:1]
    inv = 1.0 / jnp.maximum(d, 1.0)
    out_ref[...] = agg * inv + z2_ref[...]


_tc_fin = pl.pallas_call(
    _tc_fin_body,
    grid=(N // BM,),
    in_specs=[
        pl.BlockSpec((NC, BM, D), lambda m: (0, m, 0)),
        pl.BlockSpec((NC, BM, DEGW), lambda m: (0, m, 0)),
        pl.BlockSpec((BM, D), lambda m: (m, 0)),
    ],
    out_specs=pl.BlockSpec((BM, D), lambda m: (m, 0)),
    out_shape=jax.ShapeDtypeStruct((N, D), jnp.float32),
)


def kernel(x, edge_index, edge_weight, W1l, b1, W1r, W2l, b2, W2r):
    del edge_weight
    src = edge_index[0].astype(jnp.int32)
    dst = edge_index[1].astype(jnp.int32)
    pad = EP - E
    src = jnp.concatenate([src, jnp.zeros((pad,), jnp.int32)])
    dst = jnp.concatenate([dst, jnp.full((pad,), N, jnp.int32)])

    nmax = max(NCH0, NCH1)

    def split(a):
        a0 = a[:NS * EPW0].reshape(NS, NCH0, CHUNK)
        a1 = a[NS * EPW0:].reshape(NS, NCH1, CHUNK)
        a0 = jnp.pad(a0, ((0, 0), (0, nmax - NCH0), (0, 0)))
        a1 = jnp.pad(a1, ((0, 0), (0, nmax - NCH1), (0, 0)))
        return jnp.concatenate([a0, a1], axis=0)

    src = split(src)
    dst = split(dst)
    b1r = b1.reshape(1, D)
    b2r = b2.reshape(1, D)

    y1, z1 = _tc_pre(x, W1l, W1r, b1r)
    acc1, deg = _sc_segsum_deg(y1, src, dst)
    y2, z2 = _tc_mid(acc1, deg, z1, W2l, W2r, b2r)
    acc2 = _sc_segsum(y2, src, dst)
    if isinstance(acc2, (list, tuple)):
        acc2 = acc2[0]
    return _tc_fin(acc2, deg, z2)


# speedup vs baseline: 2.3808x; 2.1391x over previous
import functools

import jax
import jax.numpy as jnp
from jax import lax
from jax.experimental import pallas as pl
from jax.experimental.pallas import tpu as pltpu
from jax.experimental.pallas import tpu_sc as plsc

N = 10000
D = 128
E = 320000
NC = 2
NS = 16
NW = NC * NS
CHUNK = 128
NCH = -(-E // (NW * CHUNK))
EPW = NCH * CHUNK
EP = NW * EPW
NPAD = 10112
RT = NPAD // NS
DEGW = 16
IB = 32
NBLK = -(-NCH // IB)


def _zero_vmem_rows(ref, rows, width, dtype=jnp.float32):
    lanes = 32 if dtype == jnp.bfloat16 else 16
    z = jnp.zeros((lanes,), dtype)

    def body(i, _):
        for j in range(width // lanes):
            ref[i, pl.ds(j * lanes, lanes)] = z
        return 0

    lax.fori_loop(0, rows, body, 0, unroll=False)


def _make_sc_segsum(with_deg):
    out_type = [jax.ShapeDtypeStruct((NC, NPAD, D), jnp.bfloat16)]
    scratch = [
        pltpu.VMEM((IB, CHUNK), jnp.int32),
        pltpu.VMEM((IB, CHUNK), jnp.int32),
        pltpu.VMEM((CHUNK, D), jnp.bfloat16),
        pltpu.VMEM((CHUNK, D), jnp.bfloat16),
        pltpu.VMEM_SHARED((NPAD, D), jnp.bfloat16),
        pltpu.SemaphoreType.DMA,
        pltpu.SemaphoreType.DMA,
    ]
    if with_deg:
        out_type.append(jax.ShapeDtypeStruct((NC, NPAD, DEGW), jnp.float32))
        scratch += [
            pltpu.VMEM((CHUNK, DEGW), jnp.float32),
            pltpu.VMEM((CHUNK, DEGW), jnp.float32),
            pltpu.VMEM_SHARED((NPAD, DEGW), jnp.float32),
        ]
    mesh = plsc.VectorSubcoreMesh(core_axis_name="c", subcore_axis_name="s")

    def body(y, srcs, dsts, *refs):
        if with_deg:
            (acc_out, deg_out, idx_s, idx_d, buf, buf1, acc_sh, sem, sem1,
             ones_v, zdeg, deg_sh) = refs
        else:
            acc_out, idx_s, idx_d, buf, buf1, acc_sh, sem, sem1 = refs
        cid = lax.axis_index("c")
        sid = lax.axis_index("s")
        wid = sid * NC + cid

        _zero_vmem_rows(buf, CHUNK, D, jnp.bfloat16)
        r0 = sid * RT
        done = 0
        for t in range((RT + CHUNK - 1) // CHUNK):
            rows = min(CHUNK, RT - done)
            pltpu.sync_copy(buf.at[pl.ds(0, rows)],
                            acc_sh.at[pl.ds(r0 + done, rows)])
            done += rows
        if with_deg:
            one = jnp.ones((16,), jnp.float32)

            def fill_ones(i, _):
                ones_v[i] = one
                return 0

            lax.fori_loop(0, CHUNK, fill_ones, 0, unroll=False)
            _zero_vmem_rows(zdeg, CHUNK, DEGW)
            done = 0
            for t in range((RT + CHUNK - 1) // CHUNK):
                rows = min(CHUNK, RT - done)
                pltpu.sync_copy(zdeg.at[pl.ds(0, rows)],
                                deg_sh.at[pl.ds(r0 + done, rows)])
                done += rows
        plsc.subcore_barrier()

        def scat(b, j):
            pltpu.sync_copy(b, acc_sh.at[idx_d.at[j]], add=True)
            if with_deg:
                pltpu.sync_copy(ones_v, deg_sh.at[idx_d.at[j]], add=True)

        def run_block(nch):
            pltpu.async_copy(y.at[idx_s.at[0]], buf, sem)
            npairs = (nch - 1) // 2

            def pair(p, _):
                j0 = 2 * p
                pltpu.make_async_copy(y.at[idx_s.at[j0]], buf, sem).wait()
                pltpu.async_copy(y.at[idx_s.at[j0 + 1]], buf1, sem1)
                scat(buf, j0)
                pltpu.make_async_copy(y.at[idx_s.at[j0 + 1]], buf1,
                                      sem1).wait()
                pltpu.async_copy(y.at[idx_s.at[j0 + 2]], buf, sem)
                scat(buf1, j0 + 1)
                return 0

            lax.fori_loop(0, npairs, pair, 0, unroll=False)
            j = 2 * npairs
            pltpu.make_async_copy(y.at[idx_s.at[j]], buf, sem).wait()
            if nch % 2 == 0:
                pltpu.async_copy(y.at[idx_s.at[j + 1]], buf1, sem1)
                scat(buf, j)
                pltpu.make_async_copy(y.at[idx_s.at[j + 1]], buf1,
                                      sem1).wait()
                scat(buf1, j + 1)
            else:
                scat(buf, j)

        for blk in range(NBLK):
            nch = min(IB, NCH - blk * IB)
            pltpu.sync_copy(srcs.at[wid, pl.ds(blk * IB, nch)],
                            idx_s.at[pl.ds(0, nch)])
            pltpu.sync_copy(dsts.at[wid, pl.ds(blk * IB, nch)],
                            idx_d.at[pl.ds(0, nch)])
            run_block(nch)
        plsc.subcore_barrier()

        pltpu.sync_copy(acc_sh.at[pl.ds(r0, RT)],
                        acc_out.at[cid, pl.ds(r0, RT)])
        if with_deg:
            pltpu.sync_copy(deg_sh.at[pl.ds(r0, RT)],
                            deg_out.at[cid, pl.ds(r0, RT)])

    return pl.kernel(body, out_type=out_type, mesh=mesh,
                     scratch_types=scratch,
                     compiler_params=pltpu.CompilerParams(
                         use_tc_tiling_on_sc=False))


_sc_segsum_deg = _make_sc_segsum(with_deg=True)
_sc_segsum = _make_sc_segsum(with_deg=False)

BM = 1000
_DN = (((1,), (1,)), ((), ()))


def _tc_pre_body(x_ref, wl_ref, wr_ref, b_ref, y_ref, z_ref):
    xb = x_ref[...]
    y_ref[...] = lax.dot_general(xb, wl_ref[...], _DN,
                                 preferred_element_type=jnp.float32
                                 ).astype(jnp.bfloat16)
    z_ref[...] = lax.dot_general(xb, wr_ref[...], _DN,
                                 preferred_element_type=jnp.float32) + b_ref[...]


_tc_pre = pl.pallas_call(
    _tc_pre_body,
    grid=(N // BM,),
    in_specs=[
        pl.BlockSpec((BM, D), lambda m: (m, 0)),
        pl.BlockSpec((D, D), lambda m: (0, 0)),
        pl.BlockSpec((D, D), lambda m: (0, 0)),
        pl.BlockSpec((1, D), lambda m: (0, 0)),
    ],
    out_specs=[pl.BlockSpec((BM, D), lambda m: (m, 0)),
               pl.BlockSpec((BM, D), lambda m: (m, 0))],
    out_shape=[jax.ShapeDtypeStruct((N, D), jnp.bfloat16),
               jax.ShapeDtypeStruct((N, D), jnp.float32)],
)


def _tc_mid_body(acc_ref, deg_ref, z1_ref, wl_ref, wr_ref, b_ref,
                 y_ref, z_ref):
    agg = acc_ref[0].astype(jnp.float32) + acc_ref[1].astype(jnp.float32)
    d = deg_ref[0, :, 0:1] + deg_ref[1, :, 0:1]
    inv = 1.0 / jnp.maximum(d, 1.0)
    h = jnp.maximum(agg * inv + z1_ref[...], 0.0)
    y_ref[...] = lax.dot_general(h, wl_ref[...], _DN,
                                 preferred_element_type=jnp.float32
                                 ).astype(jnp.bfloat16)
    z_ref[...] = lax.dot_general(h, wr_ref[...], _DN,
                                 preferred_element_type=jnp.float32) + b_ref[...]


_tc_mid = pl.pallas_call(
    _tc_mid_body,
    grid=(N // BM,),
    in_specs=[
        pl.BlockSpec((NC, BM, D), lambda m: (0, m, 0)),
        pl.BlockSpec((NC, BM, DEGW), lambda m: (0, m, 0)),
        pl.BlockSpec((BM, D), lambda m: (m, 0)),
        pl.BlockSpec((D, D), lambda m: (0, 0)),
        pl.BlockSpec((D, D), lambda m: (0, 0)),
        pl.BlockSpec((1, D), lambda m: (0, 0)),
    ],
    out_specs=[pl.BlockSpec((BM, D), lambda m: (m, 0)),
               pl.BlockSpec((BM, D), lambda m: (m, 0))],
    out_shape=[jax.ShapeDtypeStruct((N, D), jnp.bfloat16),
               jax.ShapeDtypeStruct((N, D), jnp.float32)],
)


def _tc_fin_body(acc_ref, deg_ref, z2_ref, out_ref):
    agg = acc_ref[0].astype(jnp.float32) + acc_ref[1].astype(jnp.float32)
    d = deg_ref[0, :, 0:1] + deg_ref[1, :, 0:1]
    inv = 1.0 / jnp.maximum(d, 1.0)
    out_ref[...] = agg * inv + z2_ref[...]


_tc_fin = pl.pallas_call(
    _tc_fin_body,
    grid=(N // BM,),
    in_specs=[
        pl.BlockSpec((NC, BM, D), lambda m: (0, m, 0)),
        pl.BlockSpec((NC, BM, DEGW), lambda m: (0, m, 0)),
        pl.BlockSpec((BM, D), lambda m: (m, 0)),
    ],
    out_specs=pl.BlockSpec((BM, D), lambda m: (m, 0)),
    out_shape=jax.ShapeDtypeStruct((N, D), jnp.float32),
)


def kernel(x, edge_index, edge_weight, W1l, b1, W1r, W2l, b2, W2r):
    del edge_weight
    src = edge_index[0].astype(jnp.int32)
    dst = edge_index[1].astype(jnp.int32)
    pad = EP - E
    src = jnp.concatenate([src, jnp.zeros((pad,), jnp.int32)])
    dst = jnp.concatenate([dst, jnp.full((pad,), N, jnp.int32)])
    src = src.reshape(NW, NCH, CHUNK)
    dst = dst.reshape(NW, NCH, CHUNK)
    b1r = b1.reshape(1, D)
    b2r = b2.reshape(1, D)

    y1, z1 = _tc_pre(x, W1l, W1r, b1r)
    acc1, deg = _sc_segsum_deg(y1, src, dst)
    y2, z2 = _tc_mid(acc1, deg, z1, W2l, W2r, b2r)
    acc2 = _sc_segsum(y2, src, dst)
    if isinstance(acc2, (list, tuple)):
        acc2 = acc2[0]
    return _tc_fin(acc2, deg, z2)


# speedup vs baseline: 2.7264x; 1.1452x over previous
import functools

import jax
import jax.numpy as jnp
from jax import lax
from jax.experimental import pallas as pl
from jax.experimental.pallas import tpu as pltpu
from jax.experimental.pallas import tpu_sc as plsc

N = 10000
D = 128
E = 320000
NC = 2
NS = 16
NW = NC * NS
CHUNK = 128
NCH = -(-E // (NW * CHUNK))
EPW = NCH * CHUNK
EP = NW * EPW
NPAD = 10112
RT = NPAD // NS
DEGW = 16
IB = 32
NBLK = -(-NCH // IB)


def _zero_vmem_rows(ref, rows, width, dtype=jnp.float32):
    lanes = 32 if dtype == jnp.bfloat16 else 16
    z = jnp.zeros((lanes,), dtype)

    def body(i, _):
        for j in range(width // lanes):
            ref[i, pl.ds(j * lanes, lanes)] = z
        return 0

    lax.fori_loop(0, rows, body, 0, unroll=False)


def _make_sc_segsum(with_deg):
    out_type = [jax.ShapeDtypeStruct((NC, NPAD, D), jnp.bfloat16)]
    scratch = [
        pltpu.VMEM((IB, CHUNK), jnp.int32),
        pltpu.VMEM((IB, CHUNK), jnp.int32),
        pltpu.VMEM((CHUNK, D), jnp.bfloat16),
        pltpu.VMEM((CHUNK, D), jnp.bfloat16),
        pltpu.VMEM((CHUNK, D), jnp.bfloat16),
        pltpu.VMEM((CHUNK, D), jnp.bfloat16),
        pltpu.VMEM_SHARED((NPAD, D), jnp.bfloat16),
        pltpu.SemaphoreType.DMA,
        pltpu.SemaphoreType.DMA,
        pltpu.SemaphoreType.DMA,
        pltpu.SemaphoreType.DMA,
    ]
    if with_deg:
        out_type.append(jax.ShapeDtypeStruct((NC, NPAD, DEGW), jnp.float32))
        scratch += [
            pltpu.VMEM((CHUNK, DEGW), jnp.float32),
            pltpu.VMEM((CHUNK, DEGW), jnp.float32),
            pltpu.VMEM_SHARED((NPAD, DEGW), jnp.float32),
        ]
    mesh = plsc.VectorSubcoreMesh(core_axis_name="c", subcore_axis_name="s")

    def body(y, srcs, dsts, *refs):
        if with_deg:
            (acc_out, deg_out, idx_s, idx_d, buf, buf1, buf2, buf3, acc_sh,
             sem, sem1, sem2, sem3, ones_v, zdeg, deg_sh) = refs
        else:
            (acc_out, idx_s, idx_d, buf, buf1, buf2, buf3, acc_sh,
             sem, sem1, sem2, sem3) = refs
        bufs = [buf, buf1, buf2, buf3]
        sems = [sem, sem1, sem2, sem3]
        cid = lax.axis_index("c")
        sid = lax.axis_index("s")
        wid = sid * NC + cid

        _zero_vmem_rows(buf, CHUNK, D, jnp.bfloat16)
        r0 = sid * RT
        done = 0
        for t in range((RT + CHUNK - 1) // CHUNK):
            rows = min(CHUNK, RT - done)
            pltpu.sync_copy(buf.at[pl.ds(0, rows)],
                            acc_sh.at[pl.ds(r0 + done, rows)])
            done += rows
        if with_deg:
            one = jnp.ones((16,), jnp.float32)

            def fill_ones(i, _):
                ones_v[i] = one
                return 0

            lax.fori_loop(0, CHUNK, fill_ones, 0, unroll=False)
            _zero_vmem_rows(zdeg, CHUNK, DEGW)
            done = 0
            for t in range((RT + CHUNK - 1) // CHUNK):
                rows = min(CHUNK, RT - done)
                pltpu.sync_copy(zdeg.at[pl.ds(0, rows)],
                                deg_sh.at[pl.ds(r0 + done, rows)])
                done += rows
        plsc.subcore_barrier()

        def scat(b, j):
            pltpu.sync_copy(b, acc_sh.at[idx_d.at[j]], add=True)
            if with_deg:
                pltpu.sync_copy(ones_v, deg_sh.at[idx_d.at[j]], add=True)

        RING = 4

        def fire(l, j):
            pltpu.async_copy(y.at[idx_s.at[j]], bufs[l], sems[l])

        def wait(l, j):
            pltpu.make_async_copy(y.at[idx_s.at[j]], bufs[l],
                                  sems[l]).wait()

        def run_block(nch):
            assert nch % RING == 0 or nch == NCH % IB
            ngr = nch // RING
            for l in range(RING - 1):
                fire(l, l)

            def group(g, _):
                for l in range(RING):
                    j = RING * g + l
                    wait(l, j)
                    fire((l + RING - 1) % RING, j + RING - 1)
                    scat(bufs[l], j)
                return 0

            lax.fori_loop(0, max(ngr - 1, 0), group, 0, unroll=False)
            for l in range(nch - RING * max(ngr - 1, 0)):
                j = RING * max(ngr - 1, 0) + l
                wait(l % RING, j)
                if j + RING - 1 < nch:
                    fire((l + RING - 1) % RING, j + RING - 1)
                scat(bufs[l % RING], j)

        for blk in range(NBLK):
            nch = min(IB, NCH - blk * IB)
            pltpu.sync_copy(srcs.at[wid, pl.ds(blk * IB, nch)],
                            idx_s.at[pl.ds(0, nch)])
            pltpu.sync_copy(dsts.at[wid, pl.ds(blk * IB, nch)],
                            idx_d.at[pl.ds(0, nch)])
            run_block(nch)
        plsc.subcore_barrier()

        pltpu.sync_copy(acc_sh.at[pl.ds(r0, RT)],
                        acc_out.at[cid, pl.ds(r0, RT)])
        if with_deg:
            pltpu.sync_copy(deg_sh.at[pl.ds(r0, RT)],
                            deg_out.at[cid, pl.ds(r0, RT)])

    return pl.kernel(body, out_type=out_type, mesh=mesh,
                     scratch_types=scratch,
                     compiler_params=pltpu.CompilerParams(
                         use_tc_tiling_on_sc=False))


_sc_segsum_deg = _make_sc_segsum(with_deg=True)
_sc_segsum = _make_sc_segsum(with_deg=False)

BM = 1000
_DN = (((1,), (1,)), ((), ()))


def _tc_pre_body(x_ref, wl_ref, wr_ref, b_ref, y_ref, z_ref):
    xb = x_ref[...]
    y_ref[...] = lax.dot_general(xb, wl_ref[...], _DN,
                                 preferred_element_type=jnp.float32
                                 ).astype(jnp.bfloat16)
    z_ref[...] = lax.dot_general(xb, wr_ref[...], _DN,
                                 preferred_element_type=jnp.float32) + b_ref[...]


_tc_pre = pl.pallas_call(
    _tc_pre_body,
    grid=(N // BM,),
    in_specs=[
        pl.BlockSpec((BM, D), lambda m: (m, 0)),
        pl.BlockSpec((D, D), lambda m: (0, 0)),
        pl.BlockSpec((D, D), lambda m: (0, 0)),
        pl.BlockSpec((1, D), lambda m: (0, 0)),
    ],
    out_specs=[pl.BlockSpec((BM, D), lambda m: (m, 0)),
               pl.BlockSpec((BM, D), lambda m: (m, 0))],
    out_shape=[jax.ShapeDtypeStruct((N, D), jnp.bfloat16),
               jax.ShapeDtypeStruct((N, D), jnp.float32)],
)


def _tc_mid_body(acc_ref, deg_ref, z1_ref, wl_ref, wr_ref, b_ref,
                 y_ref, z_ref):
    agg = acc_ref[0].astype(jnp.float32) + acc_ref[1].astype(jnp.float32)
    d = deg_ref[0, :, 0:1] + deg_ref[1, :, 0:1]
    inv = 1.0 / jnp.maximum(d, 1.0)
    h = jnp.maximum(agg * inv + z1_ref[...], 0.0)
    y_ref[...] = lax.dot_general(h, wl_ref[...], _DN,
                                 preferred_element_type=jnp.float32
                                 ).astype(jnp.bfloat16)
    z_ref[...] = lax.dot_general(h, wr_ref[...], _DN,
                                 preferred_element_type=jnp.float32) + b_ref[...]


_tc_mid = pl.pallas_call(
    _tc_mid_body,
    grid=(N // BM,),
    in_specs=[
        pl.BlockSpec((NC, BM, D), lambda m: (0, m, 0)),
        pl.BlockSpec((NC, BM, DEGW), lambda m: (0, m, 0)),
        pl.BlockSpec((BM, D), lambda m: (m, 0)),
        pl.BlockSpec((D, D), lambda m: (0, 0)),
        pl.BlockSpec((D, D), lambda m: (0, 0)),
        pl.BlockSpec((1, D), lambda m: (0, 0)),
    ],
    out_specs=[pl.BlockSpec((BM, D), lambda m: (m, 0)),
               pl.BlockSpec((BM, D), lambda m: (m, 0))],
    out_shape=[jax.ShapeDtypeStruct((N, D), jnp.bfloat16),
               jax.ShapeDtypeStruct((N, D), jnp.float32)],
)


def _tc_fin_body(acc_ref, deg_ref, z2_ref, out_ref):
    agg = acc_ref[0].astype(jnp.float32) + acc_ref[1].astype(jnp.float32)
    d = deg_ref[0, :, 0:1] + deg_ref[1, :, 0:1]
    inv = 1.0 / jnp.maximum(d, 1.0)
    out_ref[...] = agg * inv + z2_ref[...]


_tc_fin = pl.pallas_call(
    _tc_fin_body,
    grid=(N // BM,),
    in_specs=[
        pl.BlockSpec((NC, BM, D), lambda m: (0, m, 0)),
        pl.BlockSpec((NC, BM, DEGW), lambda m: (0, m, 0)),
        pl.BlockSpec((BM, D), lambda m: (m, 0)),
    ],
    out_specs=pl.BlockSpec((BM, D), lambda m: (m, 0)),
    out_shape=jax.ShapeDtypeStruct((N, D), jnp.float32),
)


def kernel(x, edge_index, edge_weight, W1l, b1, W1r, W2l, b2, W2r):
    del edge_weight
    src = edge_index[0].astype(jnp.int32)
    dst = edge_index[1].astype(jnp.int32)
    pad = EP - E
    src = jnp.concatenate([src, jnp.zeros((pad,), jnp.int32)])
    dst = jnp.concatenate([dst, jnp.full((pad,), N, jnp.int32)])
    src = src.reshape(NW, NCH, CHUNK)
    dst = dst.reshape(NW, NCH, CHUNK)
    b1r = b1.reshape(1, D)
    b2r = b2.reshape(1, D)

    y1, z1 = _tc_pre(x, W1l, W1r, b1r)
    acc1, deg = _sc_segsum_deg(y1, src, dst)
    y2, z2 = _tc_mid(acc1, deg, z1, W2l, W2r, b2r)
    acc2 = _sc_segsum(y2, src, dst)
    if isinstance(acc2, (list, tuple)):
        acc2 = acc2[0]
    return _tc_fin(acc2, deg, z2)
